# baseline (speedup 1.0000x reference)
import functools

import jax
import jax.numpy as jnp
from jax import lax
from jax.experimental import pallas as pl
from jax.experimental.pallas import tpu as pltpu
from jax.experimental.pallas import tpu_sc as plsc

N = 16384
D = 256
K = 1024
BN = 512
GRID = N // BN


def _dist_argmin_kernel(f2_ref, c_ref, fn_ref, cn_ref, col_ref, lab_ref):
    f2 = f2_ref[...] * -2.0
    c = c_ref[...]
    mm2 = lax.dot_general(f2, c, (((1,), (1,)), ((), ())),
                          preferred_element_type=jnp.float32)
    fn = fn_ref[...].reshape(BN, 1)
    cn = cn_ref[...].reshape(1, K)
    s = (fn + cn) + mm2
    xm = jnp.min(s, axis=1, keepdims=True)
    m = jnp.sqrt(jnp.maximum(xm, 0.0))
    mi = lax.bitcast_convert_type(m, jnp.int32)
    nm = lax.bitcast_convert_type(mi + 1, jnp.float32)
    p = m * nm
    pi = lax.bitcast_convert_type(p, jnp.int32)
    lo = lax.bitcast_convert_type(pi - 1, jnp.float32)
    hi = lax.bitcast_convert_type(pi + 1, jnp.float32)
    b = jnp.where(jnp.sqrt(hi) == m, hi,
                  jnp.where(jnp.sqrt(p) == m, p,
                            jnp.where(jnp.sqrt(lo) == m, lo, xm)))
    b = jnp.where(m > 0.0, b, 0.0)
    colf = col_ref[...]
    labf = jnp.min(jnp.where(s <= b, colf, float(K)), axis=1)
    lab_ref[...] = labf.astype(jnp.int32).reshape(1, 1, BN)


def _compute_labels(f2d, centroids, fn, cn):
    nrows = f2d.shape[0]
    grid = nrows // BN
    labs = pl.pallas_call(
        _dist_argmin_kernel,
        grid=(grid,),
        in_specs=[
            pl.BlockSpec((BN, D), lambda i: (i, 0)),
            pl.BlockSpec((K, D), lambda i: (0, 0)),
            pl.BlockSpec((1, 1, BN), lambda i: (i, 0, 0)),
            pl.BlockSpec((1, K), lambda i: (0, 0)),
            pl.BlockSpec((1, K), lambda i: (0, 0)),
        ],
        out_specs=pl.BlockSpec((1, 1, BN), lambda i: (i, 0, 0)),
        out_shape=jax.ShapeDtypeStruct((grid, 1, BN), jnp.int32),
    )(f2d, centroids, fn.reshape(grid, 1, BN), cn.reshape(1, K),
      jnp.arange(K, dtype=jnp.float32).reshape(1, K))
    return labs.reshape(nrows)


_NC, _NS = 2, 16
_NW = _NC * _NS
_BPW = N // _NW
_CH = 64
_NB = 6

def _make_sc_gather_body(nrows):
    nch = nrows // _NW // _CH

    nb = min(_NB, nch)

    def body(table_hbm, idx_hbm, out_hbm, idx_v, *scr):
        wid = lax.axis_index("s") * _NC + lax.axis_index("c")
        base = wid * (nrows // _NW)
        pltpu.sync_copy(idx_hbm.at[pl.ds(wid * nch, nch)], idx_v)
        bufs = scr[:nb]
        gsems = scr[nb:2 * nb]
        osems = scr[2 * nb:3 * nb]
        gcp = [None] * nch
        ocp = [None] * nch
        for j in range(nch):
            b = j % nb
            if j >= nb:
                ocp[j - nb].wait()
            gcp[j] = pltpu.async_copy(table_hbm.at[idx_v.at[j]], bufs[b],
                                      gsems[b])
            if j >= 1:
                gcp[j - 1].wait()
                ocp[j - 1] = pltpu.async_copy(
                    bufs[(j - 1) % nb],
                    out_hbm.at[pl.ds(base + (j - 1) * _CH, _CH)],
                    osems[(j - 1) % nb])
        gcp[nch - 1].wait()
        ocp[nch - 1] = pltpu.async_copy(
            bufs[(nch - 1) % nb],
            out_hbm.at[pl.ds(base + (nch - 1) * _CH, _CH)],
            osems[(nch - 1) % nb])
        for j in range(max(0, nch - nb), nch):
            ocp[j].wait()

    return body


@functools.cache
def _sc_gather(nrows):
    mesh = plsc.VectorSubcoreMesh(core_axis_name="c", subcore_axis_name="s",
                                  num_cores=_NC, num_subcores=_NS)
    nch = nrows // _NW // _CH
    nb = min(_NB, nch)
    return pl.kernel(
        _make_sc_gather_body(nrows),
        out_type=jax.ShapeDtypeStruct((nrows, D), jnp.float32),
        mesh=mesh,
        scratch_types=(
            [pltpu.VMEM((nch, _CH), jnp.int32)]
            + [pltpu.VMEM((_CH, D), jnp.float32) for _ in range(nb)]
            + [pltpu.SemaphoreType.DMA for _ in range(2 * nb)]
        ),
    )


NSPLIT = 1


def kernel(feats, centroids):
    batch_shape = feats.shape[:-1]
    f2d = feats.reshape(-1, D)
    fn = jnp.sum(f2d ** 2, axis=-1)
    cn = jnp.sum(centroids ** 2, axis=-1)
    ns = N // NSPLIT
    labels = []
    assigned = []
    for i in range(NSPLIT):
        labels.append(_compute_labels(
            f2d[i * ns:(i + 1) * ns], centroids, fn[i * ns:(i + 1) * ns], cn))
    for i in range(NSPLIT):
        assigned.append(
            _sc_gather(ns)(centroids, labels[i].reshape(ns // _CH, _CH)))
    if NSPLIT == 1:
        return (labels[0].reshape(batch_shape),
                assigned[0].reshape(*batch_shape, D))
    labels = jnp.concatenate(labels).reshape(batch_shape)
    assigned = jnp.concatenate(assigned).reshape(*batch_shape, D)
    return labels, assigned


# speedup vs baseline: 1.0655x; 1.0655x over previous
import functools

import jax
import jax.numpy as jnp
from jax import lax
from jax.experimental import pallas as pl
from jax.experimental.pallas import tpu as pltpu
from jax.experimental.pallas import tpu_sc as plsc

N = 16384
D = 256
K = 1024
BN = 1024
GRID = N // BN


def _dist_argmin_kernel(f2_ref, c_ref, fn_ref, cn_ref, col_ref, lab_ref):
    f2 = f2_ref[...] * -2.0
    c = c_ref[...]
    mm2 = lax.dot_general(f2, c, (((1,), (1,)), ((), ())),
                          preferred_element_type=jnp.float32)
    fn = fn_ref[...].reshape(BN, 1)
    cn = cn_ref[...].reshape(1, K)
    s = (fn + cn) + mm2
    xm = jnp.min(s, axis=1, keepdims=True)
    m = jnp.sqrt(jnp.maximum(xm, 0.0))
    mi = lax.bitcast_convert_type(m, jnp.int32)
    nm = lax.bitcast_convert_type(mi + 1, jnp.float32)
    p = m * nm
    pi = lax.bitcast_convert_type(p, jnp.int32)
    lo = lax.bitcast_convert_type(pi - 1, jnp.float32)
    hi = lax.bitcast_convert_type(pi + 1, jnp.float32)
    b = jnp.where(jnp.sqrt(hi) == m, hi,
                  jnp.where(jnp.sqrt(p) == m, p,
                            jnp.where(jnp.sqrt(lo) == m, lo, xm)))
    b = jnp.where(m > 0.0, b, 0.0)
    colf = col_ref[...]
    labf = jnp.min(jnp.where(s <= b, colf, float(K)), axis=1)
    lab_ref[...] = labf.astype(jnp.int32).reshape(1, 1, BN)


def _compute_labels(f2d, centroids, fn, cn):
    nrows = f2d.shape[0]
    grid = nrows // BN
    labs = pl.pallas_call(
        _dist_argmin_kernel,
        grid=(grid,),
        in_specs=[
            pl.BlockSpec((BN, D), lambda i: (i, 0)),
            pl.BlockSpec((K, D), lambda i: (0, 0)),
            pl.BlockSpec((1, 1, BN), lambda i: (i, 0, 0)),
            pl.BlockSpec((1, K), lambda i: (0, 0)),
            pl.BlockSpec((1, K), lambda i: (0, 0)),
        ],
        out_specs=pl.BlockSpec((1, 1, BN), lambda i: (i, 0, 0)),
        out_shape=jax.ShapeDtypeStruct((grid, 1, BN), jnp.int32),
    )(f2d, centroids, fn.reshape(grid, 1, BN), cn.reshape(1, K),
      jnp.arange(K, dtype=jnp.float32).reshape(1, K))
    return labs.reshape(nrows)


_NC, _NS = 2, 16
_NW = _NC * _NS
_BPW = N // _NW
_CH = 128
_NB = 3

def _make_sc_gather_body(nrows):
    nch = nrows // _NW // _CH

    nb = min(_NB, nch)

    def body(table_hbm, idx_hbm, out_hbm, idx_v, *scr):
        wid = lax.axis_index("s") * _NC + lax.axis_index("c")
        base = wid * (nrows // _NW)
        pltpu.sync_copy(idx_hbm.at[pl.ds(wid * nch, nch)], idx_v)
        bufs = scr[:nb]
        gsems = scr[nb:2 * nb]
        osems = scr[2 * nb:3 * nb]
        gcp = [None] * nch
        ocp = [None] * nch
        for j in range(nch):
            b = j % nb
            if j >= nb:
                ocp[j - nb].wait()
            gcp[j] = pltpu.async_copy(table_hbm.at[idx_v.at[j]], bufs[b],
                                      gsems[b])
            if j >= 1:
                gcp[j - 1].wait()
                ocp[j - 1] = pltpu.async_copy(
                    bufs[(j - 1) % nb],
                    out_hbm.at[pl.ds(base + (j - 1) * _CH, _CH)],
                    osems[(j - 1) % nb])
        gcp[nch - 1].wait()
        ocp[nch - 1] = pltpu.async_copy(
            bufs[(nch - 1) % nb],
            out_hbm.at[pl.ds(base + (nch - 1) * _CH, _CH)],
            osems[(nch - 1) % nb])
        for j in range(max(0, nch - nb), nch):
            ocp[j].wait()

    return body


@functools.cache
def _sc_gather(nrows):
    mesh = plsc.VectorSubcoreMesh(core_axis_name="c", subcore_axis_name="s",
                                  num_cores=_NC, num_subcores=_NS)
    nch = nrows // _NW // _CH
    nb = min(_NB, nch)
    return pl.kernel(
        _make_sc_gather_body(nrows),
        out_type=jax.ShapeDtypeStruct((nrows, D), jnp.float32),
        mesh=mesh,
        scratch_types=(
            [pltpu.VMEM((nch, _CH), jnp.int32)]
            + [pltpu.VMEM((_CH, D), jnp.float32) for _ in range(nb)]
            + [pltpu.SemaphoreType.DMA for _ in range(2 * nb)]
        ),
    )


NSPLIT = 1


def kernel(feats, centroids):
    batch_shape = feats.shape[:-1]
    f2d = feats.reshape(-1, D)
    fn = jnp.sum(f2d ** 2, axis=-1)
    cn = jnp.sum(centroids ** 2, axis=-1)
    ns = N // NSPLIT
    labels = []
    assigned = []
    for i in range(NSPLIT):
        labels.append(_compute_labels(
            f2d[i * ns:(i + 1) * ns], centroids, fn[i * ns:(i + 1) * ns], cn))
    for i in range(NSPLIT):
        assigned.append(
            _sc_gather(ns)(centroids, labels[i].reshape(ns // _CH, _CH)))
    if NSPLIT == 1:
        return (labels[0].reshape(batch_shape),
                assigned[0].reshape(*batch_shape, D))
    labels = jnp.concatenate(labels).reshape(batch_shape)
    assigned = jnp.concatenate(assigned).reshape(*batch_shape, D)
    return labels, assigned


# speedup vs baseline: 1.1415x; 1.0713x over previous
import functools

import jax
import jax.numpy as jnp
from jax import lax
from jax.experimental import pallas as pl
from jax.experimental.pallas import tpu as pltpu
from jax.experimental.pallas import tpu_sc as plsc

N = 16384
D = 256
K = 1024
BN = 1024
GRID = N // BN


def _dist_argmin_kernel(f_ref, c_ref, cn_ref, col_ref, lab_ref):
    f = f_ref[...]
    f2 = f * -2.0
    c = c_ref[...]
    mm2 = lax.dot_general(f2, c, (((1,), (1,)), ((), ())),
                          preferred_element_type=jnp.float32)
    fn = jnp.sum(f * f, axis=1, keepdims=True)
    cn = cn_ref[...].reshape(1, K)
    s = (fn + cn) + mm2
    xm = jnp.min(s, axis=1, keepdims=True)
    m = jnp.sqrt(jnp.maximum(xm, 0.0))
    mi = lax.bitcast_convert_type(m, jnp.int32)
    nm = lax.bitcast_convert_type(mi + 1, jnp.float32)
    p = m * nm
    pi = lax.bitcast_convert_type(p, jnp.int32)
    lo = lax.bitcast_convert_type(pi - 1, jnp.float32)
    hi = lax.bitcast_convert_type(pi + 1, jnp.float32)
    b = jnp.where(jnp.sqrt(hi) == m, hi,
                  jnp.where(jnp.sqrt(p) == m, p,
                            jnp.where(jnp.sqrt(lo) == m, lo, xm)))
    b = jnp.where(m > 0.0, b, 0.0)
    colf = col_ref[...]
    labf = jnp.min(jnp.where(s <= b, colf, float(K)), axis=1)
    lab_ref[...] = labf.astype(jnp.int32).reshape(1, 1, BN)


def _compute_labels(f2d, centroids, cn):
    nrows = f2d.shape[0]
    grid = nrows // BN
    labs = pl.pallas_call(
        _dist_argmin_kernel,
        grid=(grid,),
        in_specs=[
            pl.BlockSpec((BN, D), lambda i: (i, 0)),
            pl.BlockSpec((K, D), lambda i: (0, 0)),
            pl.BlockSpec((1, K), lambda i: (0, 0)),
            pl.BlockSpec((1, K), lambda i: (0, 0)),
        ],
        out_specs=pl.BlockSpec((1, 1, BN), lambda i: (i, 0, 0)),
        out_shape=jax.ShapeDtypeStruct((grid, 1, BN), jnp.int32),
    )(f2d, centroids, cn.reshape(1, K),
      jnp.arange(K, dtype=jnp.float32).reshape(1, K))
    return labs.reshape(nrows)


_NC, _NS = 2, 16
_NW = _NC * _NS
_BPW = N // _NW
_CH = 128
_NB = 3

def _make_sc_gather_body(nrows):
    nch = nrows // _NW // _CH

    nb = min(_NB, nch)

    def body(table_hbm, idx_hbm, out_hbm, idx_v, *scr):
        wid = lax.axis_index("s") * _NC + lax.axis_index("c")
        base = wid * (nrows // _NW)
        pltpu.sync_copy(idx_hbm.at[pl.ds(wid * nch, nch)], idx_v)
        bufs = scr[:nb]
        gsems = scr[nb:2 * nb]
        osems = scr[2 * nb:3 * nb]
        gcp = [None] * nch
        ocp = [None] * nch
        for j in range(nch):
            b = j % nb
            if j >= nb:
                ocp[j - nb].wait()
            gcp[j] = pltpu.async_copy(table_hbm.at[idx_v.at[j]], bufs[b],
                                      gsems[b])
            if j >= 1:
                gcp[j - 1].wait()
                ocp[j - 1] = pltpu.async_copy(
                    bufs[(j - 1) % nb],
                    out_hbm.at[pl.ds(base + (j - 1) * _CH, _CH)],
                    osems[(j - 1) % nb])
        gcp[nch - 1].wait()
        ocp[nch - 1] = pltpu.async_copy(
            bufs[(nch - 1) % nb],
            out_hbm.at[pl.ds(base + (nch - 1) * _CH, _CH)],
            osems[(nch - 1) % nb])
        for j in range(max(0, nch - nb), nch):
            ocp[j].wait()

    return body


@functools.cache
def _sc_gather(nrows):
    mesh = plsc.VectorSubcoreMesh(core_axis_name="c", subcore_axis_name="s",
                                  num_cores=_NC, num_subcores=_NS)
    nch = nrows // _NW // _CH
    nb = min(_NB, nch)
    return pl.kernel(
        _make_sc_gather_body(nrows),
        out_type=jax.ShapeDtypeStruct((nrows, D), jnp.float32),
        mesh=mesh,
        scratch_types=(
            [pltpu.VMEM((nch, _CH), jnp.int32)]
            + [pltpu.VMEM((_CH, D), jnp.float32) for _ in range(nb)]
            + [pltpu.SemaphoreType.DMA for _ in range(2 * nb)]
        ),
    )


NSPLIT = 1


def kernel(feats, centroids):
    batch_shape = feats.shape[:-1]
    f2d = feats.reshape(-1, D)
    cn = jnp.sum(centroids ** 2, axis=-1)
    ns = N // NSPLIT
    labels = []
    assigned = []
    for i in range(NSPLIT):
        labels.append(_compute_labels(f2d[i * ns:(i + 1) * ns], centroids, cn))
    for i in range(NSPLIT):
        assigned.append(
            _sc_gather(ns)(centroids, labels[i].reshape(ns // _CH, _CH)))
    if NSPLIT == 1:
        return (labels[0].reshape(batch_shape),
                assigned[0].reshape(*batch_shape, D))
    labels = jnp.concatenate(labels).reshape(batch_shape)
    assigned = jnp.concatenate(assigned).reshape(*batch_shape, D)
    return labels, assigned


# speedup vs baseline: 1.1818x; 1.0353x over previous
import functools

import jax
import jax.numpy as jnp
from jax import lax
from jax.experimental import pallas as pl
from jax.experimental.pallas import tpu as pltpu
from jax.experimental.pallas import tpu_sc as plsc

N = 16384
D = 256
K = 1024
BN = 2048
GRID = N // BN


def _dist_argmin_kernel(f_ref, c_ref, cn_ref, col_ref, lab_ref):
    f = f_ref[...]
    f2 = f * -2.0
    c = c_ref[...]
    mm2 = lax.dot_general(f2, c, (((1,), (1,)), ((), ())),
                          preferred_element_type=jnp.float32)
    fn = jnp.sum(f * f, axis=1, keepdims=True)
    cn = cn_ref[...].reshape(1, K)
    s = (fn + cn) + mm2
    xm = jnp.min(s, axis=1, keepdims=True)
    m = jnp.sqrt(jnp.maximum(xm, 0.0))
    mi = lax.bitcast_convert_type(m, jnp.int32)
    nm = lax.bitcast_convert_type(mi + 1, jnp.float32)
    p = m * nm
    pi = lax.bitcast_convert_type(p, jnp.int32)
    lo = lax.bitcast_convert_type(pi - 1, jnp.float32)
    hi = lax.bitcast_convert_type(pi + 1, jnp.float32)
    b = jnp.where(jnp.sqrt(hi) == m, hi,
                  jnp.where(jnp.sqrt(p) == m, p,
                            jnp.where(jnp.sqrt(lo) == m, lo, xm)))
    b = jnp.where(m > 0.0, b, 0.0)
    colf = col_ref[...]
    labf = jnp.min(jnp.where(s <= b, colf, float(K)), axis=1)
    lab_ref[...] = labf.astype(jnp.int32).reshape(1, 1, BN)


def _compute_labels(f2d, centroids, cn):
    nrows = f2d.shape[0]
    grid = nrows // BN
    labs = pl.pallas_call(
        _dist_argmin_kernel,
        grid=(grid,),
        in_specs=[
            pl.BlockSpec((BN, D), lambda i: (i, 0)),
            pl.BlockSpec((K, D), lambda i: (0, 0)),
            pl.BlockSpec((1, K), lambda i: (0, 0)),
            pl.BlockSpec((1, K), lambda i: (0, 0)),
        ],
        out_specs=pl.BlockSpec((1, 1, BN), lambda i: (i, 0, 0)),
        out_shape=jax.ShapeDtypeStruct((grid, 1, BN), jnp.int32),
    )(f2d, centroids, cn.reshape(1, K),
      jnp.arange(K, dtype=jnp.float32).reshape(1, K))
    return labs.reshape(nrows)


_NC, _NS = 2, 16
_NW = _NC * _NS
_BPW = N // _NW
_CH = 128
_NB = 3

def _make_sc_gather_body(nrows):
    nch = nrows // _NW // _CH

    nb = min(_NB, nch)

    def body(table_hbm, idx_hbm, out_hbm, idx_v, *scr):
        wid = lax.axis_index("s") * _NC + lax.axis_index("c")
        base = wid * (nrows // _NW)
        pltpu.sync_copy(idx_hbm.at[pl.ds(wid * nch, nch)], idx_v)
        bufs = scr[:nb]
        gsems = scr[nb:2 * nb]
        osems = scr[2 * nb:3 * nb]
        gcp = [None] * nch
        ocp = [None] * nch
        for j in range(nch):
            b = j % nb
            if j >= nb:
                ocp[j - nb].wait()
            gcp[j] = pltpu.async_copy(table_hbm.at[idx_v.at[j]], bufs[b],
                                      gsems[b])
            if j >= 1:
                gcp[j - 1].wait()
                ocp[j - 1] = pltpu.async_copy(
                    bufs[(j - 1) % nb],
                    out_hbm.at[pl.ds(base + (j - 1) * _CH, _CH)],
                    osems[(j - 1) % nb])
        gcp[nch - 1].wait()
        ocp[nch - 1] = pltpu.async_copy(
            bufs[(nch - 1) % nb],
            out_hbm.at[pl.ds(base + (nch - 1) * _CH, _CH)],
            osems[(nch - 1) % nb])
        for j in range(max(0, nch - nb), nch):
            ocp[j].wait()

    return body


@functools.cache
def _sc_gather(nrows):
    mesh = plsc.VectorSubcoreMesh(core_axis_name="c", subcore_axis_name="s",
                                  num_cores=_NC, num_subcores=_NS)
    nch = nrows // _NW // _CH
    nb = min(_NB, nch)
    return pl.kernel(
        _make_sc_gather_body(nrows),
        out_type=jax.ShapeDtypeStruct((nrows, D), jnp.float32),
        mesh=mesh,
        scratch_types=(
            [pltpu.VMEM((nch, _CH), jnp.int32)]
            + [pltpu.VMEM((_CH, D), jnp.float32) for _ in range(nb)]
            + [pltpu.SemaphoreType.DMA for _ in range(2 * nb)]
        ),
    )


NSPLIT = 1


def kernel(feats, centroids):
    batch_shape = feats.shape[:-1]
    f2d = feats.reshape(-1, D)
    cn = jnp.sum(centroids ** 2, axis=-1)
    ns = N // NSPLIT
    labels = []
    assigned = []
    for i in range(NSPLIT):
        labels.append(_compute_labels(f2d[i * ns:(i + 1) * ns], centroids, cn))
    for i in range(NSPLIT):
        assigned.append(
            _sc_gather(ns)(centroids, labels[i].reshape(ns // _CH, _CH)))
    if NSPLIT == 1:
        return (labels[0].reshape(batch_shape),
                assigned[0].reshape(*batch_shape, D))
    labels = jnp.concatenate(labels).reshape(batch_shape)
    assigned = jnp.concatenate(assigned).reshape(*batch_shape, D)
    return labels, assigned


# speedup vs baseline: 1.1996x; 1.0150x over previous
import functools

import jax
import jax.numpy as jnp
from jax import lax
from jax.experimental import pallas as pl
from jax.experimental.pallas import tpu as pltpu
from jax.experimental.pallas import tpu_sc as plsc

N = 16384
D = 256
K = 1024
BN = 4096
GRID = N // BN


def _dist_argmin_kernel(f_ref, c_ref, cn_ref, col_ref, lab_ref):
    f = f_ref[...]
    f2 = f * -2.0
    c = c_ref[...]
    mm2 = lax.dot_general(f2, c, (((1,), (1,)), ((), ())),
                          preferred_element_type=jnp.float32)
    fn = jnp.sum(f * f, axis=1, keepdims=True)
    cn = cn_ref[...].reshape(1, K)
    s = (fn + cn) + mm2
    xm = jnp.min(s, axis=1, keepdims=True)
    m = jnp.sqrt(jnp.maximum(xm, 0.0))
    mi = lax.bitcast_convert_type(m, jnp.int32)
    nm = lax.bitcast_convert_type(mi + 1, jnp.float32)
    p = m * nm
    pi = lax.bitcast_convert_type(p, jnp.int32)
    lo = lax.bitcast_convert_type(pi - 1, jnp.float32)
    hi = lax.bitcast_convert_type(pi + 1, jnp.float32)
    b = jnp.where(jnp.sqrt(hi) == m, hi,
                  jnp.where(jnp.sqrt(p) == m, p,
                            jnp.where(jnp.sqrt(lo) == m, lo, xm)))
    b = jnp.where(m > 0.0, b, 0.0)
    colf = col_ref[...]
    labf = jnp.min(jnp.where(s <= b, colf, float(K)), axis=1)
    lab_ref[...] = labf.astype(jnp.int32).reshape(1, 1, BN)


def _compute_labels(f2d, centroids, cn):
    nrows = f2d.shape[0]
    grid = nrows // BN
    labs = pl.pallas_call(
        _dist_argmin_kernel,
        grid=(grid,),
        in_specs=[
            pl.BlockSpec((BN, D), lambda i: (i, 0)),
            pl.BlockSpec((K, D), lambda i: (0, 0)),
            pl.BlockSpec((1, K), lambda i: (0, 0)),
            pl.BlockSpec((1, K), lambda i: (0, 0)),
        ],
        out_specs=pl.BlockSpec((1, 1, BN), lambda i: (i, 0, 0)),
        out_shape=jax.ShapeDtypeStruct((grid, 1, BN), jnp.int32),
    )(f2d, centroids, cn.reshape(1, K),
      jnp.arange(K, dtype=jnp.float32).reshape(1, K))
    return labs.reshape(nrows)


_NC, _NS = 2, 16
_NW = _NC * _NS
_BPW = N // _NW
_CH = 128
_NB = 3

def _make_sc_gather_body(nrows):
    nch = nrows // _NW // _CH

    nb = min(_NB, nch)

    def body(table_hbm, idx_hbm, out_hbm, idx_v, *scr):
        wid = lax.axis_index("s") * _NC + lax.axis_index("c")
        base = wid * (nrows // _NW)
        pltpu.sync_copy(idx_hbm.at[pl.ds(wid * nch, nch)], idx_v)
        bufs = scr[:nb]
        gsems = scr[nb:2 * nb]
        osems = scr[2 * nb:3 * nb]
        gcp = [None] * nch
        ocp = [None] * nch
        for j in range(nch):
            b = j % nb
            if j >= nb:
                ocp[j - nb].wait()
            gcp[j] = pltpu.async_copy(table_hbm.at[idx_v.at[j]], bufs[b],
                                      gsems[b])
            if j >= 1:
                gcp[j - 1].wait()
                ocp[j - 1] = pltpu.async_copy(
                    bufs[(j - 1) % nb],
                    out_hbm.at[pl.ds(base + (j - 1) * _CH, _CH)],
                    osems[(j - 1) % nb])
        gcp[nch - 1].wait()
        ocp[nch - 1] = pltpu.async_copy(
            bufs[(nch - 1) % nb],
            out_hbm.at[pl.ds(base + (nch - 1) * _CH, _CH)],
            osems[(nch - 1) % nb])
        for j in range(max(0, nch - nb), nch):
            ocp[j].wait()

    return body


@functools.cache
def _sc_gather(nrows):
    mesh = plsc.VectorSubcoreMesh(core_axis_name="c", subcore_axis_name="s",
                                  num_cores=_NC, num_subcores=_NS)
    nch = nrows // _NW // _CH
    nb = min(_NB, nch)
    return pl.kernel(
        _make_sc_gather_body(nrows),
        out_type=jax.ShapeDtypeStruct((nrows, D), jnp.float32),
        mesh=mesh,
        scratch_types=(
            [pltpu.VMEM((nch, _CH), jnp.int32)]
            + [pltpu.VMEM((_CH, D), jnp.float32) for _ in range(nb)]
            + [pltpu.SemaphoreType.DMA for _ in range(2 * nb)]
        ),
    )


NSPLIT = 1


def kernel(feats, centroids):
    batch_shape = feats.shape[:-1]
    f2d = feats.reshape(-1, D)
    cn = jnp.sum(centroids ** 2, axis=-1)
    ns = N // NSPLIT
    labels = []
    assigned = []
    for i in range(NSPLIT):
        labels.append(_compute_labels(f2d[i * ns:(i + 1) * ns], centroids, cn))
    for i in range(NSPLIT):
        assigned.append(
            _sc_gather(ns)(centroids, labels[i].reshape(ns // _CH, _CH)))
    if NSPLIT == 1:
        return (labels[0].reshape(batch_shape),
                assigned[0].reshape(*batch_shape, D))
    labels = jnp.concatenate(labels).reshape(batch_shape)
    assigned = jnp.concatenate(assigned).reshape(*batch_shape, D)
    return labels, assigned
